# Initial kernel scaffold; baseline (speedup 1.0000x reference)
import jax
import jax.numpy as jnp
from jax.experimental import pallas as pl


def kernel(cosine, label):
    raise NotImplementedError("write your pallas kernel here")


# speedup vs baseline: 1.2700x; 1.2700x over previous
import jax
import jax.numpy as jnp
from jax import lax
from jax.experimental import pallas as pl
from jax.experimental.pallas import tpu as pltpu
from jax.experimental.pallas import tpu_sc as plsc

_SCALE = 64.0
_MARGIN = 0.35
_B = 1024
_V = 100000
_BR = 8

_NC = 2
_NS = 16
_NW = _NC * _NS
_BPW = _B // _NW


def _sc_gather_body(flat_hbm, label_hbm, t_hbm, lab_v, idx_v, t_v, sem):
    wid = lax.axis_index("s") * _NC + lax.axis_index("c")
    base = wid * _BPW
    pltpu.sync_copy(label_hbm.at[pl.ds(base, _BPW)], lab_v)
    row0 = base * _V
    for j in range(_BPW // 16):
        lab = lab_v[pl.ds(j * 16, 16)]
        idx = lab + (lax.iota(jnp.int32, 16) * _V + (row0 + j * 16 * _V))
        idx_v[pl.ds(j * 16, 16)] = idx
    pltpu.async_copy(flat_hbm.at[idx_v], t_v, sem).wait()
    pltpu.sync_copy(t_v, t_hbm.at[pl.ds(base, _BPW)])


def _sc_gather(cosine, label):
    mesh = plsc.VectorSubcoreMesh(core_axis_name="c", subcore_axis_name="s")
    return pl.kernel(
        _sc_gather_body,
        out_type=jax.ShapeDtypeStruct((_B,), jnp.float32),
        mesh=mesh,
        scratch_types=[
            pltpu.VMEM((_BPW,), jnp.int32),
            pltpu.VMEM((_BPW,), jnp.int32),
            pltpu.VMEM((_BPW,), jnp.float32),
            pltpu.SemaphoreType.DMA,
        ],
    )(cosine.reshape(_B * _V), label.astype(jnp.int32))


def _tc_reduce_body(t_ref, cos_ref, out_ref):
    x = cos_ref[...]
    m = jnp.max(x, axis=1, keepdims=True)
    s = jnp.sum(jnp.exp((x - m) * _SCALE), axis=1, keepdims=True)
    mm = m * _SCALE
    t64 = t_ref[...] * _SCALE
    delta = _SCALE * _MARGIN
    sp = s - jnp.exp(t64 - mm) * (1.0 - jnp.exp(jnp.float32(-delta)))
    nll = mm + jnp.log(sp) - t64 + delta
    part = jnp.sum(nll, keepdims=True) * (1.0 / _B)
    prev = jnp.where(pl.program_id(0) == 0, jnp.zeros((1, 1), jnp.float32),
                     out_ref[...])
    out_ref[...] = prev + part


def _tc_loss(t, cosine):
    out = pl.pallas_call(
        _tc_reduce_body,
        grid=(_B // _BR,),
        in_specs=[
            pl.BlockSpec((_BR, 1), lambda i: (i, 0)),
            pl.BlockSpec((_BR, _V), lambda i: (i, 0)),
        ],
        out_specs=pl.BlockSpec((1, 1), lambda i: (0, 0)),
        out_shape=jax.ShapeDtypeStruct((1, 1), jnp.float32),
    )(t.reshape(_B, 1), cosine)
    return out[0, 0]


def kernel(cosine, label):
    t = _sc_gather(cosine, label)
    return _tc_loss(t, cosine)


# speedup vs baseline: 1.3958x; 1.0991x over previous
import jax
import jax.numpy as jnp
from jax import lax
from jax.experimental import pallas as pl
from jax.experimental.pallas import tpu as pltpu
from jax.experimental.pallas import tpu_sc as plsc

_SCALE = 64.0
_MARGIN = 0.35
_B = 1024
_V = 100000
_BR = 32

_NC = 2
_NS = 16
_NW = _NC * _NS
_BPW = _B // _NW


def _sc_gather_body(flat_hbm, label_hbm, t_hbm, lab_v, idx_v, t_v, sem):
    wid = lax.axis_index("s") * _NC + lax.axis_index("c")
    base = wid * _BPW
    pltpu.sync_copy(label_hbm.at[pl.ds(base, _BPW)], lab_v)
    row0 = base * _V
    for j in range(_BPW // 16):
        lab = lab_v[pl.ds(j * 16, 16)]
        idx = lab + (lax.iota(jnp.int32, 16) * _V + (row0 + j * 16 * _V))
        idx_v[pl.ds(j * 16, 16)] = idx
    pltpu.async_copy(flat_hbm.at[idx_v], t_v, sem).wait()
    pltpu.sync_copy(t_v, t_hbm.at[pl.ds(base, _BPW)])


def _sc_gather(cosine, label):
    mesh = plsc.VectorSubcoreMesh(core_axis_name="c", subcore_axis_name="s")
    return pl.kernel(
        _sc_gather_body,
        out_type=jax.ShapeDtypeStruct((_B,), jnp.float32),
        mesh=mesh,
        scratch_types=[
            pltpu.VMEM((_BPW,), jnp.int32),
            pltpu.VMEM((_BPW,), jnp.int32),
            pltpu.VMEM((_BPW,), jnp.float32),
            pltpu.SemaphoreType.DMA,
        ],
    )(cosine.reshape(_B * _V), label.astype(jnp.int32))


def _tc_reduce_body(t_ref, cos_ref, out_ref):
    x = cos_ref[...]
    m = jnp.max(x, axis=1, keepdims=True)
    s = jnp.sum(jnp.exp((x - m) * _SCALE), axis=1, keepdims=True)
    mm = m * _SCALE
    t64 = t_ref[...] * _SCALE
    delta = _SCALE * _MARGIN
    sp = s - jnp.exp(t64 - mm) * (1.0 - jnp.exp(jnp.float32(-delta)))
    nll = mm + jnp.log(sp) - t64 + delta
    part = jnp.sum(nll, keepdims=True) * (1.0 / _B)
    prev = jnp.where(pl.program_id(0) == 0, jnp.zeros((1, 1), jnp.float32),
                     out_ref[...])
    out_ref[...] = prev + part


def _tc_loss(t, cosine):
    out = pl.pallas_call(
        _tc_reduce_body,
        grid=(_B // _BR,),
        in_specs=[
            pl.BlockSpec((_BR, 1), lambda i: (i, 0)),
            pl.BlockSpec((_BR, _V), lambda i: (i, 0)),
        ],
        out_specs=pl.BlockSpec((1, 1), lambda i: (0, 0)),
        out_shape=jax.ShapeDtypeStruct((1, 1), jnp.float32),
    )(t.reshape(_B, 1), cosine)
    return out[0, 0]


def kernel(cosine, label):
    t = _sc_gather(cosine, label)
    return _tc_loss(t, cosine)


# speedup vs baseline: 1.4127x; 1.0121x over previous
import jax
import jax.numpy as jnp
from jax import lax
from jax.experimental import pallas as pl
from jax.experimental.pallas import tpu as pltpu
from jax.experimental.pallas import tpu_sc as plsc

_SCALE = 64.0
_MARGIN = 0.35
_B = 1024
_V = 100000
_BR = 32

_NC = 2
_NS = 16
_NW = _NC * _NS
_BPW = _B // _NW


def _sc_gather_body(flat_hbm, label_hbm, t_hbm, lab_v, idx_v, t_v, sem):
    wid = lax.axis_index("s") * _NC + lax.axis_index("c")
    base = wid * _BPW
    pltpu.sync_copy(label_hbm.at[pl.ds(base, _BPW)], lab_v)
    row0 = base * _V
    for j in range(_BPW // 16):
        lab = lab_v[pl.ds(j * 16, 16)]
        idx = lab + (lax.iota(jnp.int32, 16) * _V + (row0 + j * 16 * _V))
        idx_v[pl.ds(j * 16, 16)] = idx
    pltpu.async_copy(flat_hbm.at[idx_v], t_v, sem).wait()
    pltpu.sync_copy(t_v, t_hbm.at[pl.ds(base, _BPW)])


def _sc_gather(cosine, label):
    mesh = plsc.VectorSubcoreMesh(core_axis_name="c", subcore_axis_name="s")
    return pl.kernel(
        _sc_gather_body,
        out_type=jax.ShapeDtypeStruct((_B,), jnp.float32),
        mesh=mesh,
        scratch_types=[
            pltpu.VMEM((_BPW,), jnp.int32),
            pltpu.VMEM((_BPW,), jnp.int32),
            pltpu.VMEM((_BPW,), jnp.float32),
            pltpu.SemaphoreType.DMA,
        ],
    )(cosine.reshape(_B * _V), label.astype(jnp.int32))


_K = 4
_VK = 25088
_TAIL = _V - (_K - 1) * _VK


def _tc_reduce_body(t_ref, c0, c1, c2, c3, out_ref):
    x3 = c3[...]
    col = lax.broadcasted_iota(jnp.int32, x3.shape, 1)
    x3 = jnp.where(col < _TAIL, x3, -jnp.inf)
    parts = (c0[...], c1[...], c2[...], x3)
    m = jnp.max(parts[0], axis=1, keepdims=True)
    for p in parts[1:]:
        m = jnp.maximum(m, jnp.max(p, axis=1, keepdims=True))
    s = jnp.zeros_like(m)
    for p in parts:
        s = s + jnp.sum(jnp.exp((p - m) * _SCALE), axis=1, keepdims=True)
    mm = m * _SCALE
    t64 = t_ref[...] * _SCALE
    delta = _SCALE * _MARGIN
    sp = s - jnp.exp(t64 - mm) * (1.0 - jnp.exp(jnp.float32(-delta)))
    nll = mm + jnp.log(sp) - t64 + delta
    part = jnp.sum(nll, keepdims=True) * (1.0 / _B)
    prev = jnp.where(pl.program_id(0) == 0, jnp.zeros((1, 1), jnp.float32),
                     out_ref[...])
    out_ref[...] = prev + part


def _tc_loss(t, cosine):
    col_specs = [
        pl.BlockSpec((_BR, _VK), lambda i, k=k: (i, k)) for k in range(_K)
    ]
    out = pl.pallas_call(
        _tc_reduce_body,
        grid=(_B // _BR,),
        in_specs=[pl.BlockSpec((_BR, 1), lambda i: (i, 0))] + col_specs,
        out_specs=pl.BlockSpec((1, 1), lambda i: (0, 0)),
        out_shape=jax.ShapeDtypeStruct((1, 1), jnp.float32),
    )(t.reshape(_B, 1), cosine, cosine, cosine, cosine)
    return out[0, 0]


def kernel(cosine, label):
    t = _sc_gather(cosine, label)
    return _tc_loss(t, cosine)
